# baseline (speedup 1.0000x reference)
import functools

import jax
import jax.numpy as jnp
from jax.experimental import pallas as pl
from jax.experimental.pallas import tpu as pltpu

_TOP_K = 2
_N_EXPERTS = 8
_HIDDEN = 768
_ALPHA = 0.001

_ROWS_PER_BLOCK = 4096


def _gate_body(hs_ref, wT_ref, idx_ref, w_ref, aux_ref, ce_acc, sc_acc,
               *, blocks_per_batch, n_blocks, aux_scale):
    pid = pl.program_id(0)

    @pl.when(pid == 0)
    def _init():
        ce_acc[...] = jnp.zeros_like(ce_acc)
        sc_acc[...] = jnp.zeros_like(sc_acc)

    x = hs_ref[...]
    logits = jnp.dot(x, wT_ref[...], preferred_element_type=jnp.float32)
    lt = logits.T

    erow = jax.lax.broadcasted_iota(jnp.int32, lt.shape, 0)
    big = jnp.int32(_N_EXPERTS)
    m1 = jnp.max(lt, axis=0, keepdims=True)
    i1 = jnp.min(jnp.where(lt == m1, erow, big), axis=0, keepdims=True)
    masked = jnp.where(erow == i1, -jnp.inf, lt)
    m2 = jnp.max(masked, axis=0, keepdims=True)
    i2 = jnp.min(jnp.where(masked == m2, erow, big), axis=0, keepdims=True)

    e = jnp.exp(lt - m1)
    z = jnp.sum(e, axis=0, keepdims=True)
    s2r = jnp.exp(m2 - m1)
    denom = 1.0 + s2r + 1e-20 * z
    w1 = 1.0 / denom
    w2 = s2r / denom

    idx_ref[0:1, :] = i1
    idx_ref[1:2, :] = i2
    w_ref[0:1, :] = w1
    w_ref[1:2, :] = w2

    b = pid // blocks_per_batch
    scores_sum = jnp.sum(e * (1.0 / z), axis=1, keepdims=True)
    cnt = jnp.sum((erow == i1).astype(jnp.float32)
                  + (erow == i2).astype(jnp.float32), axis=1, keepdims=True)
    bcol = (jax.lax.broadcasted_iota(jnp.int32, ce_acc.shape, 1)
            == b).astype(jnp.float32)
    ce_acc[...] += bcol * cnt
    sc_acc[...] += bcol * scores_sum

    @pl.when(pid == n_blocks - 1)
    def _finish():
        aux_ref[...] = jnp.sum(ce_acc[...] * sc_acc[...],
                               keepdims=True) * aux_scale


def kernel(hidden_states, weight):
    bsz, seq_len, h = hidden_states.shape
    n_tok = bsz * seq_len
    hs_flat = hidden_states.reshape(n_tok, h)
    wT = weight.T

    rows = _ROWS_PER_BLOCK
    n_blocks = n_tok // rows
    blocks_per_batch = seq_len // rows
    aux_scale = (_N_EXPERTS / (seq_len * _TOP_K)) / seq_len / bsz * _ALPHA

    body = functools.partial(
        _gate_body,
        blocks_per_batch=blocks_per_batch,
        n_blocks=n_blocks,
        aux_scale=aux_scale,
    )

    idx_t, w_t, aux = pl.pallas_call(
        body,
        grid=(n_blocks,),
        in_specs=[
            pl.BlockSpec((rows, h), lambda i: (i, 0)),
            pl.BlockSpec((h, _N_EXPERTS), lambda i: (0, 0)),
        ],
        out_specs=[
            pl.BlockSpec((_TOP_K, rows), lambda i: (0, i)),
            pl.BlockSpec((_TOP_K, rows), lambda i: (0, i)),
            pl.BlockSpec((1, 1), lambda i: (0, 0)),
        ],
        out_shape=[
            jax.ShapeDtypeStruct((_TOP_K, n_tok), jnp.int32),
            jax.ShapeDtypeStruct((_TOP_K, n_tok), jnp.float32),
            jax.ShapeDtypeStruct((1, 1), jnp.float32),
        ],
        scratch_shapes=[
            pltpu.VMEM((_N_EXPERTS, bsz), jnp.float32),
            pltpu.VMEM((_N_EXPERTS, bsz), jnp.float32),
        ],
    )(hs_flat, wT)

    return idx_t.T, w_t.T, aux[0, 0]
